# baseline (speedup 1.0000x reference)
import functools

import jax
import jax.numpy as jnp
from jax import lax
from jax.experimental import pallas as pl
from jax.experimental.pallas import tpu as pltpu
from jax.experimental.pallas import tpu_sc as plsc

_K = 8192
_D = 32
_T = 512

_NC = 2
_NS = 16
_NW = _NC * _NS


def _argmin_body(xb, xsqb, wsq, w2, oidx):
    mm2 = jax.lax.dot_general(
        xb[...], w2[...], (((1,), (1,)), ((), ())),
        preferred_element_type=jnp.float32)
    dist = (xsqb[...] + wsq[...]) - mm2
    rowmin = jnp.min(dist, axis=1, keepdims=True)
    iota = jax.lax.broadcasted_iota(jnp.int32, dist.shape, 1)
    oidx[...] = jnp.min(jnp.where(dist == rowmin, iota, jnp.int32(_K)),
                        axis=1, keepdims=True)


_DP = 128


def _make_gather(n):
    b_per_w = n // _NW
    mesh = plsc.VectorSubcoreMesh(core_axis_name="c", subcore_axis_name="s")

    @functools.partial(
        pl.kernel,
        out_type=jax.ShapeDtypeStruct((n, _DP), jnp.float32),
        mesh=mesh,
        scratch_types=[
            pltpu.VMEM((b_per_w,), jnp.int32),
            pltpu.VMEM((b_per_w, _DP), jnp.float32),
            pltpu.SemaphoreType.DMA,
        ],
    )
    def _gather(table_hbm, idx_hbm, out_hbm, idx_v, rows_v, sem):
        wid = lax.axis_index("s") * _NC + lax.axis_index("c")
        base = wid * b_per_w
        pltpu.sync_copy(idx_hbm.at[pl.ds(base, b_per_w)], idx_v)
        pltpu.async_copy(table_hbm.at[idx_v], rows_v, sem).wait()
        pltpu.sync_copy(rows_v, out_hbm.at[pl.ds(base, b_per_w)])

    return _gather


def kernel(x, W):
    b, c, h, w = x.shape
    xp = jnp.transpose(x, (0, 2, 3, 1)).reshape(-1, c)
    n = xp.shape[0]
    x_sq = jnp.sum(xp * xp, axis=1, keepdims=True)
    w_sq = jnp.sum(W * W, axis=1)[None, :]
    w2 = W * 2.0

    idx = pl.pallas_call(
        _argmin_body,
        grid=(n // _T,),
        in_specs=[
            pl.BlockSpec((_T, _D), lambda i: (i, 0)),
            pl.BlockSpec((_T, 1), lambda i: (i, 0)),
            pl.BlockSpec((1, _K), lambda i: (0, 0)),
            pl.BlockSpec((_K, _D), lambda i: (0, 0)),
        ],
        out_specs=pl.BlockSpec((_T, 1), lambda i: (i, 0)),
        out_shape=jax.ShapeDtypeStruct((n, 1), jnp.int32),
    )(xp, x_sq, w_sq, w2)

    w_pad = jnp.pad(W, ((0, 0), (0, _DP - _D)))
    quant = _make_gather(n)(w_pad, idx.reshape(n))[:, :_D]

    q4 = quant.reshape(b, h, w, c).transpose(0, 3, 1, 2)
    return x + (q4 - x)


# speedup vs baseline: 1.1426x; 1.1426x over previous
import functools

import jax
import jax.numpy as jnp
from jax import lax
from jax.experimental import pallas as pl
from jax.experimental.pallas import tpu as pltpu
from jax.experimental.pallas import tpu_sc as plsc

_K = 8192
_D = 32
_A = 64
_T = 512

_NC = 2
_NS = 16
_NW = _NC * _NS

_DP = 128


def _argmin_body(xa, wa, oidx):
    s = jax.lax.dot_general(
        xa[...], wa[...], (((1,), (1,)), ((), ())),
        preferred_element_type=jnp.float32)
    rowmin = jnp.min(s, axis=1, keepdims=True)
    iota = jax.lax.broadcasted_iota(jnp.int32, s.shape, 1)
    oidx[...] = jnp.min(jnp.where(s == rowmin, iota, jnp.int32(_K)),
                        axis=1, keepdims=True)


def _make_gather(n):
    b_per_w = n // _NW
    mesh = plsc.VectorSubcoreMesh(core_axis_name="c", subcore_axis_name="s")

    @functools.partial(
        pl.kernel,
        out_type=jax.ShapeDtypeStruct((n, _DP), jnp.float32),
        mesh=mesh,
        scratch_types=[
            pltpu.VMEM((b_per_w,), jnp.int32),
            pltpu.VMEM((b_per_w, _DP), jnp.float32),
            pltpu.SemaphoreType.DMA,
        ],
    )
    def _gather(table_hbm, idx_hbm, out_hbm, idx_v, rows_v, sem):
        wid = lax.axis_index("s") * _NC + lax.axis_index("c")
        base = wid * b_per_w
        pltpu.sync_copy(idx_hbm.at[pl.ds(base, b_per_w)], idx_v)
        pltpu.async_copy(table_hbm.at[idx_v], rows_v, sem).wait()
        pltpu.sync_copy(rows_v, out_hbm.at[pl.ds(base, b_per_w)])

    return _gather


def kernel(x, W):
    b, c, h, w = x.shape
    xp = jnp.transpose(x, (0, 2, 3, 1)).reshape(-1, c)
    n = xp.shape[0]
    w_sq = jnp.sum(W * W, axis=1, keepdims=True)
    x_aug = jnp.concatenate(
        [xp, jnp.ones((n, 1), jnp.float32),
         jnp.zeros((n, _A - _D - 1), jnp.float32)], axis=1)
    w_aug = jnp.concatenate(
        [W * -2.0, w_sq, jnp.zeros((_K, _A - _D - 1), jnp.float32)], axis=1)

    idx = pl.pallas_call(
        _argmin_body,
        grid=(n // _T,),
        in_specs=[
            pl.BlockSpec((_T, _A), lambda i: (i, 0)),
            pl.BlockSpec((_K, _A), lambda i: (0, 0)),
        ],
        out_specs=pl.BlockSpec((_T, 1), lambda i: (i, 0)),
        out_shape=jax.ShapeDtypeStruct((n, 1), jnp.int32),
    )(x_aug, w_aug)

    w_pad = jnp.pad(W, ((0, 0), (0, _DP - _D)))
    quant = _make_gather(n)(w_pad, idx.reshape(n))[:, :_D]

    q4 = quant.reshape(b, h, w, c).transpose(0, 3, 1, 2)
    return x + (q4 - x)


# speedup vs baseline: 1.1678x; 1.0221x over previous
import functools

import jax
import jax.numpy as jnp
from jax import lax
from jax.experimental import pallas as pl
from jax.experimental.pallas import tpu as pltpu
from jax.experimental.pallas import tpu_sc as plsc

_K = 8192
_D = 32
_A = 64
_T = 1024

_NC = 2
_NS = 16
_NW = _NC * _NS

_DP = 128


def _argmin_body(xa, wa, oidx):
    s = jax.lax.dot_general(
        xa[...], wa[...], (((1,), (1,)), ((), ())),
        preferred_element_type=jnp.float32)
    rowmin = jnp.min(s, axis=1, keepdims=True)
    iota = jax.lax.broadcasted_iota(jnp.int32, s.shape, 1)
    oidx[...] = jnp.min(jnp.where(s == rowmin, iota, jnp.int32(_K)),
                        axis=1, keepdims=True)


def _make_gather(n):
    b_per_w = n // _NW
    mesh = plsc.VectorSubcoreMesh(core_axis_name="c", subcore_axis_name="s")

    @functools.partial(
        pl.kernel,
        out_type=jax.ShapeDtypeStruct((n, _DP), jnp.float32),
        mesh=mesh,
        scratch_types=[
            pltpu.VMEM((b_per_w,), jnp.int32),
            pltpu.VMEM((b_per_w, _DP), jnp.float32),
            pltpu.SemaphoreType.DMA,
        ],
    )
    def _gather(table_hbm, idx_hbm, out_hbm, idx_v, rows_v, sem):
        wid = lax.axis_index("s") * _NC + lax.axis_index("c")
        base = wid * b_per_w
        pltpu.sync_copy(idx_hbm.at[pl.ds(base, b_per_w)], idx_v)
        pltpu.async_copy(table_hbm.at[idx_v], rows_v, sem).wait()
        pltpu.sync_copy(rows_v, out_hbm.at[pl.ds(base, b_per_w)])

    return _gather


def kernel(x, W):
    b, c, h, w = x.shape
    xp = jnp.transpose(x, (0, 2, 3, 1)).reshape(-1, c)
    n = xp.shape[0]
    w_sq = jnp.sum(W * W, axis=1, keepdims=True)
    x_aug = jnp.concatenate(
        [xp, jnp.ones((n, 1), jnp.float32),
         jnp.zeros((n, _A - _D - 1), jnp.float32)], axis=1)
    w_aug = jnp.concatenate(
        [W * -2.0, w_sq, jnp.zeros((_K, _A - _D - 1), jnp.float32)], axis=1)

    idx = pl.pallas_call(
        _argmin_body,
        grid=(n // _T,),
        in_specs=[
            pl.BlockSpec((_T, _A), lambda i: (i, 0)),
            pl.BlockSpec((_K, _A), lambda i: (0, 0)),
        ],
        out_specs=pl.BlockSpec((_T, 1), lambda i: (i, 0)),
        out_shape=jax.ShapeDtypeStruct((n, 1), jnp.int32),
    )(x_aug, w_aug)

    w_pad = jnp.pad(W, ((0, 0), (0, _DP - _D)))
    quant = _make_gather(n)(w_pad, idx.reshape(n))[:, :_D]

    q4 = quant.reshape(b, h, w, c).transpose(0, 3, 1, 2)
    return x + (q4 - x)
